# Initial kernel scaffold; baseline (speedup 1.0000x reference)
import jax
import jax.numpy as jnp
from jax.experimental import pallas as pl


def kernel(idx, E, l, h, r):
    raise NotImplementedError("write your pallas kernel here")


# speedup vs baseline: 2.5431x; 2.5431x over previous
import functools

import jax
import jax.numpy as jnp
from jax import lax
from jax.experimental import pallas as pl
from jax.experimental.pallas import tpu as pltpu
from jax.experimental.pallas import tpu_sc as plsc

_CH = 128


def _weight_body(r_ref, l_ref, h_ref, e_ref, o_ref):
    r = r_ref[...]
    o_ref[...] = r * l_ref[...] + (1.0 - r) * h_ref[...] + e_ref[...]


@functools.lru_cache(maxsize=None)
def _gather_call(V, D, N, NC, NS):
    NW = NC * NS
    n_per_w = N // NW
    nch = n_per_w // _CH
    mesh = plsc.VectorSubcoreMesh(core_axis_name="c", subcore_axis_name="s")

    @functools.partial(
        pl.kernel,
        mesh=mesh,
        out_type=jax.ShapeDtypeStruct((N, D), jnp.float32),
        scratch_types=[
            pltpu.VMEM((nch, _CH), jnp.int32),
            pltpu.VMEM((_CH, D), jnp.float32),
            pltpu.SemaphoreType.DMA,
        ],
    )
    def gather_k(table_hbm, idx_hbm, out_hbm, idx_v, buf, gsem):
        wid = lax.axis_index("s") * NC + lax.axis_index("c")
        base = wid * n_per_w
        pltpu.sync_copy(idx_hbm.at[wid], idx_v)

        def body(j, carry):
            pltpu.async_copy(table_hbm.at[idx_v.at[j]], buf, gsem).wait()
            pltpu.sync_copy(buf, out_hbm.at[pl.ds(base + j * _CH, _CH)])
            return carry

        lax.fori_loop(0, nch, body, 0)

    return gather_k


def kernel(idx, E, l, h, r):
    B, F = idx.shape
    V, D = E.shape
    N = B * F

    weight = pl.pallas_call(
        _weight_body,
        out_shape=jax.ShapeDtypeStruct((V, D), jnp.float32),
    )(r, l, h, E)

    info = plsc.get_sparse_core_info()
    NC, NS = info.num_cores, info.num_subcores
    NW = NC * NS
    n_per_w = N // NW
    idx3 = idx.reshape(-1).astype(jnp.int32).reshape(NW, n_per_w // _CH, _CH)
    out = _gather_call(V, D, N, NC, NS)(weight, idx3)
    return out.reshape(B, F, D)


# speedup vs baseline: 2.6019x; 1.0231x over previous
import functools

import jax
import jax.numpy as jnp
from jax import lax
from jax.experimental import pallas as pl
from jax.experimental.pallas import tpu as pltpu
from jax.experimental.pallas import tpu_sc as plsc

_CH = 104
_NBUF = 4


def _weight_body(r_ref, l_ref, h_ref, e_ref, o_ref):
    r = r_ref[...]
    o_ref[...] = r * l_ref[...] + (1.0 - r) * h_ref[...] + e_ref[...]


@functools.lru_cache(maxsize=None)
def _gather_call(V, D, N, NC, NS):
    NW = NC * NS
    n_per_w = N // NW
    nch = n_per_w // _CH
    assert nch % _NBUF == 0
    nouter = nch // _NBUF
    mesh = plsc.VectorSubcoreMesh(core_axis_name="c", subcore_axis_name="s")

    @functools.partial(
        pl.kernel,
        mesh=mesh,
        out_type=jax.ShapeDtypeStruct((N, D), jnp.float32),
        scratch_types=(
            [pltpu.VMEM((nch, _CH), jnp.int32)]
            + [pltpu.VMEM((_CH, D), jnp.float32) for _ in range(_NBUF)]
            + [pltpu.SemaphoreType.DMA for _ in range(2 * _NBUF)]
        ),
    )
    def gather_k(table_hbm, idx_hbm, out_hbm, idx_v, *rest):
        bufs = rest[:_NBUF]
        gsems = rest[_NBUF:2 * _NBUF]
        wsems = rest[2 * _NBUF:]
        wid = lax.axis_index("s") * NC + lax.axis_index("c")
        base = wid * n_per_w
        pltpu.sync_copy(idx_hbm.at[wid], idx_v)

        def issue_gather(j, b):
            pltpu.async_copy(table_hbm.at[idx_v.at[j]], bufs[b], gsems[b])

        def wait_gather(b):
            pltpu.make_async_copy(
                table_hbm.at[pl.ds(0, _CH)], bufs[b], gsems[b]).wait()

        def issue_write(j, b):
            pltpu.async_copy(
                bufs[b], out_hbm.at[pl.ds(base + j * _CH, _CH)], wsems[b])

        def wait_write(b):
            pltpu.make_async_copy(
                bufs[b], out_hbm.at[pl.ds(0, _CH)], wsems[b]).wait()

        for b in range(_NBUF):
            issue_gather(b, b)

        def body(i, carry):
            j0 = i * _NBUF
            for b in range(_NBUF):
                j = j0 + b
                wait_gather(b)
                issue_write(j, b)

                @pl.when(j + _NBUF < nch)
                def _():
                    wait_write(b)
                    issue_gather(j + _NBUF, b)
            return carry

        lax.fori_loop(0, nouter, body, 0)
        for b in range(_NBUF):
            wait_write(b)

    return gather_k


def kernel(idx, E, l, h, r):
    B, F = idx.shape
    V, D = E.shape
    N = B * F

    weight = pl.pallas_call(
        _weight_body,
        out_shape=jax.ShapeDtypeStruct((V, D), jnp.float32),
    )(r, l, h, E)

    info = plsc.get_sparse_core_info()
    NC, NS = info.num_cores, info.num_subcores
    NW = NC * NS
    n_per_w = N // NW
    idx3 = idx.reshape(-1).astype(jnp.int32).reshape(NW, n_per_w // _CH, _CH)
    out = _gather_call(V, D, N, NC, NS)(weight, idx3)
    return out.reshape(B, F, D)
